# Initial kernel scaffold; baseline (speedup 1.0000x reference)
import functools

import jax
import jax.numpy as jnp
from jax import lax
from jax.experimental import pallas as pl
from jax.experimental.pallas import tpu as pltpu
from jax.experimental.pallas import tpu_sc as plsc

B = 4096
T = 50
NSF = 26
NDF = 5
V = 100000
D = 16
NRS = 4
NRD = 8
COUT = NSF * D + NRS + NDF * D + NRD

NW = 32
BPW = B // NW
G = 8
DYN_CHUNK = G * T * NDF
DYN_PAD = 2048
NCH = BPW // G
STAT_ROWS = BPW * NSF


def _sc_gather(stat_idx, dyn_idx, ws_flat, wd_flat):
    mesh = plsc.VectorSubcoreMesh(core_axis_name="c", subcore_axis_name="s")

    @functools.partial(
        pl.kernel,
        mesh=mesh,
        out_type=(
            jax.ShapeDtypeStruct((B * NSF, D), jnp.float32),
            jax.ShapeDtypeStruct((B * T * NDF, D), jnp.float32),
        ),
        scratch_types=[
            pltpu.VMEM((STAT_ROWS,), jnp.int32),
            pltpu.VMEM((NSF, 128), jnp.int32),
            pltpu.VMEM((STAT_ROWS, D), jnp.float32),
            pltpu.VMEM((DYN_PAD,), jnp.int32),
            pltpu.VMEM((16, 128), jnp.int32),
            pltpu.VMEM((DYN_PAD, D), jnp.float32),
            pltpu.SemaphoreType.DMA,
        ],
    )
    def k(stat_idx_h, dyn_idx_h, ws_h, wd_h, out_s_h, out_d_h,
          raw_s, idx_s, rows_s, raw_d, idx_d, rows_d, sem):
        wid = lax.axis_index("s") * 2 + lax.axis_index("c")
        iota = lax.iota(jnp.int32, 16)

        z = jnp.zeros((16,), jnp.int32)
        raw_d[pl.ds(DYN_CHUNK, 16)] = z
        raw_d[pl.ds(DYN_CHUNK + 16, 16)] = z
        raw_d[pl.ds(DYN_CHUNK + 32, 16)] = z

        sb = wid * STAT_ROWS
        pltpu.sync_copy(stat_idx_h.at[pl.ds(sb, STAT_ROWS)], raw_s)

        def stat_comp(r, carry):
            for c8 in range(8):
                off = r * 128 + c8 * 16
                p = off + iota
                idx_s[r, pl.ds(c8 * 16, 16)] = (
                    raw_s[pl.ds(off, 16)] + (p % NSF) * V)
            return carry

        lax.fori_loop(0, NSF, stat_comp, 0)

        hs = [
            pltpu.async_copy(ws_h.at[idx_s.at[j]],
                             rows_s.at[pl.ds(j * 128, 128)], sem)
            for j in range(NSF)
        ]
        for h in hs:
            h.wait()
        pltpu.sync_copy(rows_s, out_s_h.at[pl.ds(sb, STAT_ROWS)])

        def dyn_body(g, carry):
            db = (wid * BPW + g * G) * T * NDF
            pltpu.sync_copy(dyn_idx_h.at[pl.ds(db, DYN_CHUNK)],
                            raw_d.at[pl.ds(0, DYN_CHUNK)])

            def dyn_comp(r, c):
                for c8 in range(8):
                    off = r * 128 + c8 * 16
                    p = off + iota
                    idx_d[r, pl.ds(c8 * 16, 16)] = (
                        raw_d[pl.ds(off, 16)] + (p % NDF) * V)
                return c

            lax.fori_loop(0, 16, dyn_comp, 0)

            hd = [
                pltpu.async_copy(wd_h.at[idx_d.at[j]],
                                 rows_d.at[pl.ds(j * 128, 128)], sem)
                for j in range(16)
            ]
            for h in hd:
                h.wait()
            pltpu.sync_copy(rows_d.at[pl.ds(0, DYN_CHUNK)],
                            out_d_h.at[pl.ds(db, DYN_CHUNK)])
            return carry

        lax.fori_loop(0, NCH, dyn_body, 0)

    return k(stat_idx, dyn_idx, ws_flat, wd_flat)


BB = 16


def _tc_assemble(stat_emb, stat_real, dyn_emb, dyn_real):
    def body(se_ref, sr_ref, de_ref, dr_ref, o_ref):
        stat = jnp.concatenate([se_ref[...], sr_ref[...]], axis=-1)
        statb = jnp.broadcast_to(stat[:, None, :], (BB, T, NSF * D + NRS))
        o_ref[...] = jnp.concatenate(
            [statb, de_ref[...], dr_ref[...]], axis=-1)

    return pl.pallas_call(
        body,
        grid=(B // BB,),
        in_specs=[
            pl.BlockSpec((BB, NSF * D), lambda i: (i, 0)),
            pl.BlockSpec((BB, NRS), lambda i: (i, 0)),
            pl.BlockSpec((BB, T, NDF * D), lambda i: (i, 0, 0)),
            pl.BlockSpec((BB, T, NRD), lambda i: (i, 0, 0)),
        ],
        out_specs=pl.BlockSpec((BB, T, COUT), lambda i: (i, 0, 0)),
        out_shape=jax.ShapeDtypeStruct((B, T, COUT), jnp.float32),
    )(stat_emb, stat_real, dyn_emb, dyn_real)


def kernel(feat_static_cat, feat_static_real, feat_dynamic_cat,
           feat_dynamic_real, W_static, W_dynamic):
    fsc = feat_static_cat.astype(jnp.int32).reshape(B * NSF)
    fdc = feat_dynamic_cat.astype(jnp.int32).reshape(B * T * NDF)
    rows_s, rows_d = _sc_gather(
        fsc, fdc,
        W_static.reshape(NSF * V, D),
        W_dynamic.reshape(NDF * V, D),
    )
    return _tc_assemble(
        rows_s.reshape(B, NSF * D),
        feat_static_real,
        rows_d.reshape(B, T, NDF * D),
        feat_dynamic_real,
    )


# speedup vs baseline: 2.8966x; 2.8966x over previous
import functools

import jax
import jax.numpy as jnp
from jax import lax
from jax.experimental import pallas as pl
from jax.experimental.pallas import tpu as pltpu
from jax.experimental.pallas import tpu_sc as plsc

B = 4096
T = 50
NSF = 26
NDF = 5
V = 100000
D = 16
NRS = 4
NRD = 8
COUT = NSF * D + NRS + NDF * D + NRD

NW = 32
BPW = B // NW
G = 8
DYN_CHUNK = G * T * NDF
DYN_PAD = 2048
NCH = BPW // G
STAT_ROWS = BPW * NSF


def _sc_gather(stat_idx, dyn_idx, ws_flat, wd_flat):
    mesh = plsc.VectorSubcoreMesh(core_axis_name="c", subcore_axis_name="s")

    @functools.partial(
        pl.kernel,
        mesh=mesh,
        compiler_params=pltpu.CompilerParams(use_tc_tiling_on_sc=False),
        out_type=(
            jax.ShapeDtypeStruct((B * NSF, D), jnp.float32),
            jax.ShapeDtypeStruct((B * T * NDF, D), jnp.float32),
        ),
        scratch_types=[
            pltpu.VMEM((STAT_ROWS,), jnp.int32),
            pltpu.VMEM((NSF, 128), jnp.int32),
            pltpu.VMEM((STAT_ROWS, D), jnp.float32),
            pltpu.VMEM((DYN_PAD,), jnp.int32),
            pltpu.VMEM((16, 128), jnp.int32),
            pltpu.VMEM((DYN_PAD, D), jnp.float32),
            pltpu.SemaphoreType.DMA,
        ],
    )
    def k(stat_idx_h, dyn_idx_h, ws_h, wd_h, out_s_h, out_d_h,
          raw_s, idx_s, rows_s, raw_d, idx_d, rows_d, sem):
        wid = lax.axis_index("s") * 2 + lax.axis_index("c")
        iota = lax.iota(jnp.int32, 16)

        z = jnp.zeros((16,), jnp.int32)
        raw_d[pl.ds(DYN_CHUNK, 16)] = z
        raw_d[pl.ds(DYN_CHUNK + 16, 16)] = z
        raw_d[pl.ds(DYN_CHUNK + 32, 16)] = z

        sb = wid * STAT_ROWS
        pltpu.sync_copy(stat_idx_h.at[pl.ds(sb, STAT_ROWS)], raw_s)

        def stat_comp(r, carry):
            for c8 in range(8):
                off = r * 128 + c8 * 16
                p = off + iota
                idx_s[r, pl.ds(c8 * 16, 16)] = (
                    raw_s[pl.ds(off, 16)] + (p % NSF) * V)
            return carry

        lax.fori_loop(0, NSF, stat_comp, 0)

        hs = [
            pltpu.async_copy(ws_h.at[idx_s.at[j]],
                             rows_s.at[pl.ds(j * 128, 128)], sem)
            for j in range(NSF)
        ]
        for h in hs:
            h.wait()
        pltpu.sync_copy(rows_s, out_s_h.at[pl.ds(sb, STAT_ROWS)])

        def dyn_body(g, carry):
            db = (wid * BPW + g * G) * T * NDF
            pltpu.sync_copy(dyn_idx_h.at[pl.ds(db, DYN_CHUNK)],
                            raw_d.at[pl.ds(0, DYN_CHUNK)])

            def dyn_comp(r, c):
                for c8 in range(8):
                    off = r * 128 + c8 * 16
                    p = off + iota
                    idx_d[r, pl.ds(c8 * 16, 16)] = (
                        raw_d[pl.ds(off, 16)] + (p % NDF) * V)
                return c

            lax.fori_loop(0, 16, dyn_comp, 0)

            hd = [
                pltpu.async_copy(wd_h.at[idx_d.at[j]],
                                 rows_d.at[pl.ds(j * 128, 128)], sem)
                for j in range(16)
            ]
            for h in hd:
                h.wait()
            pltpu.sync_copy(rows_d.at[pl.ds(0, DYN_CHUNK)],
                            out_d_h.at[pl.ds(db, DYN_CHUNK)])
            return carry

        lax.fori_loop(0, NCH, dyn_body, 0)

    return k(stat_idx, dyn_idx, ws_flat, wd_flat)


BB = 16


def _tc_assemble(stat_emb, stat_real, dyn_emb, dyn_real):
    def body(se_ref, sr_ref, de_ref, dr_ref, o_ref):
        stat = jnp.concatenate([se_ref[...], sr_ref[...]], axis=-1)
        statb = jnp.broadcast_to(stat[:, None, :], (BB, T, NSF * D + NRS))
        o_ref[...] = jnp.concatenate(
            [statb, de_ref[...], dr_ref[...]], axis=-1)

    return pl.pallas_call(
        body,
        grid=(B // BB,),
        in_specs=[
            pl.BlockSpec((BB, NSF * D), lambda i: (i, 0)),
            pl.BlockSpec((BB, NRS), lambda i: (i, 0)),
            pl.BlockSpec((BB, T, NDF * D), lambda i: (i, 0, 0)),
            pl.BlockSpec((BB, T, NRD), lambda i: (i, 0, 0)),
        ],
        out_specs=pl.BlockSpec((BB, T, COUT), lambda i: (i, 0, 0)),
        out_shape=jax.ShapeDtypeStruct((B, T, COUT), jnp.float32),
    )(stat_emb, stat_real, dyn_emb, dyn_real)


def kernel(feat_static_cat, feat_static_real, feat_dynamic_cat,
           feat_dynamic_real, W_static, W_dynamic):
    fsc = feat_static_cat.astype(jnp.int32).reshape(B * NSF)
    fdc = feat_dynamic_cat.astype(jnp.int32).reshape(B * T * NDF)
    rows_s, rows_d = _sc_gather(
        fsc, fdc,
        W_static.reshape(NSF * V, D),
        W_dynamic.reshape(NDF * V, D),
    )
    return _tc_assemble(
        rows_s.reshape(B, NSF * D),
        feat_static_real,
        rows_d.reshape(B, T, NDF * D),
        feat_dynamic_real,
    )


# speedup vs baseline: 6.0776x; 2.0982x over previous
import functools

import jax
import jax.numpy as jnp
from jax import lax
from jax.experimental import pallas as pl
from jax.experimental.pallas import tpu as pltpu
from jax.experimental.pallas import tpu_sc as plsc

B = 4096
T = 50
NSF = 26
NDF = 5
V = 100000
D = 16
NRS = 4
NRD = 8
CS = NSF * D
COUT = CS + NRS + NDF * D + NRD

NW = 32
BPW = B // NW
SCB = 32
SROWS = SCB * CS
NSCH = BPW // SCB

RPW = (T * B) // NW
DCR = 400
DROWS = DCR * NDF
DPAD = 2048
NDCH = RPW // DCR


def _sc_gather(fsc_t, fdc_seg, ws_cols, wd_flat):
    mesh = plsc.VectorSubcoreMesh(core_axis_name="c", subcore_axis_name="s")

    @functools.partial(
        pl.kernel,
        mesh=mesh,
        compiler_params=pltpu.CompilerParams(
            use_tc_tiling_on_sc=False, needs_layout_passes=False),
        out_type=(
            jax.ShapeDtypeStruct((B * CS,), jnp.float32),
            jax.ShapeDtypeStruct((T * B * NDF, D), jnp.float32),
        ),
        scratch_types=[
            pltpu.VMEM((NSF * BPW,), jnp.int32),
            pltpu.VMEM((SROWS // 128, 128), jnp.int32),
            pltpu.VMEM((SROWS,), jnp.float32),
            pltpu.VMEM((NDF * DCR + 48,), jnp.int32),
            pltpu.VMEM((DPAD // 128, 128), jnp.int32),
            pltpu.VMEM((DPAD, D), jnp.float32),
            pltpu.SemaphoreType.DMA,
        ],
    )
    def k(fsc_h, fdc_h, wsc_h, wd_h, outs_h, outd_h,
          sbuf, sidx, sdst, dbuf, didx, ddst, sem):
        w = lax.axis_index("s") * 2 + lax.axis_index("c")
        iota = lax.iota(jnp.int32, 16)
        b0 = w * BPW
        r0 = w * RPW

        hs = [
            pltpu.async_copy(fsc_h.at[pl.ds(i * B + b0, BPW)],
                             sbuf.at[pl.ds(i * BPW, BPW)], sem)
            for i in range(NSF)
        ]
        for h in hs:
            h.wait()

        def s_chunk(ci, carry):
            def comp(q, c2):
                p = q * 16 + iota
                col = p % CS
                bl = ci * SCB + p // CS
                raw = plsc.load_gather(sbuf, [(col // D) * BPW + bl])
                sidx[q // 8, pl.ds((q % 8) * 16, 16)] = col * V + raw
                return c2
            lax.fori_loop(0, SROWS // 16, comp, 0)

            def s_gat(j, c2):
                pltpu.async_copy(wsc_h.at[sidx.at[j]],
                                 sdst.at[pl.ds(j * 128, 128)], sem)
                return c2
            lax.fori_loop(0, SROWS // 128, s_gat, 0)
            pltpu.make_async_copy(wsc_h.at[pl.ds(0, SROWS)], sdst,
                                  sem).wait()
            pltpu.async_copy(
                sdst, outs_h.at[pl.ds((b0 + ci * SCB) * CS, SROWS)],
                sem).wait()
            return carry
        lax.fori_loop(0, NSCH, s_chunk, 0)

        def d_chunk(ci, carry):
            off = ci * DCR
            hseg = [
                pltpu.async_copy(fdc_h.at[f, pl.ds(r0 + off, DCR)],
                                 dbuf.at[pl.ds(f * DCR, DCR)], sem)
                for f in range(NDF)
            ]
            for h in hseg:
                h.wait()

            def comp(q, c2):
                p = q * 16 + iota
                rr = jnp.minimum(p // NDF, DCR - 1)
                f = p % NDF
                raw = plsc.load_gather(dbuf, [f * DCR + rr])
                didx[q // 8, pl.ds((q % 8) * 16, 16)] = f * V + raw
                return c2
            lax.fori_loop(0, DPAD // 16, comp, 0)

            def d_gat(j, c2):
                pltpu.async_copy(wd_h.at[didx.at[j]],
                                 ddst.at[pl.ds(j * 128, 128)], sem)
                return c2
            lax.fori_loop(0, DPAD // 128, d_gat, 0)
            pltpu.make_async_copy(wd_h.at[pl.ds(0, DPAD)], ddst,
                                  sem).wait()
            pltpu.async_copy(
                ddst.at[pl.ds(0, DROWS)],
                outd_h.at[pl.ds((r0 + off) * NDF, DROWS)], sem).wait()
            return carry
        lax.fori_loop(0, NDCH, d_chunk, 0)

    return k(fsc_t, fdc_seg, ws_cols, wd_flat)


BB = 16


def _tc_assemble(stat_emb, stat_real, dyn_emb, dyn_real):
    def body(se_ref, sr_ref, de_ref, dr_ref, o_ref):
        stat = jnp.concatenate([se_ref[...], sr_ref[...]], axis=-1)
        statb = jnp.broadcast_to(stat[None, :, :], (T, BB, CS + NRS))
        o_ref[...] = jnp.concatenate(
            [statb, de_ref[...], dr_ref[...]], axis=-1)

    return pl.pallas_call(
        body,
        grid=(B // BB,),
        in_specs=[
            pl.BlockSpec((BB, CS), lambda i: (i, 0)),
            pl.BlockSpec((BB, NRS), lambda i: (i, 0)),
            pl.BlockSpec((T, BB, NDF * D), lambda i: (0, i, 0)),
            pl.BlockSpec((T, BB, NRD), lambda i: (0, i, 0)),
        ],
        out_specs=pl.BlockSpec((T, BB, COUT), lambda i: (0, i, 0)),
        out_shape=jax.ShapeDtypeStruct((T, B, COUT), jnp.float32),
    )(stat_emb, stat_real, dyn_emb, dyn_real)


def kernel(feat_static_cat, feat_static_real, feat_dynamic_cat,
           feat_dynamic_real, W_static, W_dynamic):
    ws_cols = jnp.transpose(W_static, (0, 2, 1)).reshape(NSF * D * V)
    wd_flat = W_dynamic.reshape(NDF * V, D)
    fsc_t = jnp.transpose(feat_static_cat.astype(jnp.int32),
                          (1, 0)).reshape(NSF * B)
    fdc_seg = jnp.transpose(feat_dynamic_cat.astype(jnp.int32),
                            (2, 1, 0)).reshape(NDF, T * B)
    out_stat, out_dyn = _sc_gather(fsc_t, fdc_seg, ws_cols, wd_flat)
    fdr_t = jnp.transpose(feat_dynamic_real, (1, 0, 2))
    out_t = _tc_assemble(
        out_stat.reshape(B, CS),
        feat_static_real,
        out_dyn.reshape(T, B, NDF * D),
        fdr_t,
    )
    return jnp.transpose(out_t, (1, 0, 2))


# speedup vs baseline: 6.6779x; 1.0988x over previous
import functools

import jax
import jax.numpy as jnp
from jax import lax
from jax.experimental import pallas as pl
from jax.experimental.pallas import tpu as pltpu
from jax.experimental.pallas import tpu_sc as plsc

B = 4096
T = 50
NSF = 26
NDF = 5
V = 100000
D = 16
NRS = 4
NRD = 8
CS = NSF * D
COUT = CS + NRS + NDF * D + NRD

NW = 32
BPW = B // NW
SCB = 32
SROWS = SCB * CS
NSCH = BPW // SCB

RPW = (T * B) // NW
DCR = 400
DROWS = DCR * NDF
DPAD = 2048
NDCH = RPW // DCR


def _sc_gather(fsc_t, fdc_seg, ws_cols, wd_flat):
    mesh = plsc.VectorSubcoreMesh(core_axis_name="c", subcore_axis_name="s")

    @functools.partial(
        pl.kernel,
        mesh=mesh,
        compiler_params=pltpu.CompilerParams(
            use_tc_tiling_on_sc=False, needs_layout_passes=False),
        out_type=(
            jax.ShapeDtypeStruct((B * CS,), jnp.float32),
            jax.ShapeDtypeStruct((T * B * NDF, D), jnp.float32),
        ),
        scratch_types=[
            pltpu.VMEM((NSF * BPW,), jnp.int32),
            pltpu.VMEM((SROWS // 128, 128), jnp.int32),
            pltpu.VMEM((SROWS,), jnp.float32),
            pltpu.VMEM((NDF * DCR + 48,), jnp.int32),
            pltpu.VMEM((DPAD // 128, 128), jnp.int32),
            pltpu.VMEM((DPAD, D), jnp.float32),
            pltpu.SemaphoreType.DMA,
        ],
    )
    def k(fsc_h, fdc_h, wsc_h, wd_h, outs_h, outd_h,
          sbuf, sidx, sdst, dbuf, didx, ddst, sem):
        w = lax.axis_index("s") * 2 + lax.axis_index("c")
        iota = lax.iota(jnp.int32, 16)
        b0 = w * BPW
        r0 = w * RPW

        hs = [
            pltpu.async_copy(fsc_h.at[pl.ds(i * B + b0, BPW)],
                             sbuf.at[pl.ds(i * BPW, BPW)], sem)
            for i in range(NSF)
        ]
        for h in hs:
            h.wait()

        def s_chunk(ci, carry):
            def comp(q, c2):
                p = q * 16 + iota
                col = p % CS
                bl = ci * SCB + p // CS
                raw = plsc.load_gather(sbuf, [(col // D) * BPW + bl])
                sidx[q // 8, pl.ds((q % 8) * 16, 16)] = col * V + raw
                return c2
            lax.fori_loop(0, SROWS // 16, comp, 0)

            def s_gat(j, c2):
                pltpu.async_copy(wsc_h.at[sidx.at[j]],
                                 sdst.at[pl.ds(j * 128, 128)], sem)
                return c2
            lax.fori_loop(0, SROWS // 128, s_gat, 0)
            pltpu.make_async_copy(wsc_h.at[pl.ds(0, SROWS)], sdst,
                                  sem).wait()
            pltpu.async_copy(
                sdst, outs_h.at[pl.ds((b0 + ci * SCB) * CS, SROWS)],
                sem).wait()
            return carry
        lax.fori_loop(0, NSCH, s_chunk, 0)

        def d_chunk(ci, carry):
            off = ci * DCR
            hseg = [
                pltpu.async_copy(fdc_h.at[f, pl.ds(r0 + off, DCR)],
                                 dbuf.at[pl.ds(f * DCR, DCR)], sem)
                for f in range(NDF)
            ]
            for h in hseg:
                h.wait()

            def comp(q, c2):
                p = q * 16 + iota
                rr = jnp.minimum(p // NDF, DCR - 1)
                f = p % NDF
                raw = plsc.load_gather(dbuf, [f * DCR + rr])
                didx[q // 8, pl.ds((q % 8) * 16, 16)] = f * V + raw
                return c2
            lax.fori_loop(0, DPAD // 16, comp, 0)

            def d_gat(j, c2):
                pltpu.async_copy(wd_h.at[didx.at[j]],
                                 ddst.at[pl.ds(j * 128, 128)], sem)
                return c2
            lax.fori_loop(0, DPAD // 128, d_gat, 0)
            pltpu.make_async_copy(wd_h.at[pl.ds(0, DPAD)], ddst,
                                  sem).wait()
            pltpu.async_copy(
                ddst.at[pl.ds(0, DROWS)],
                outd_h.at[pl.ds((r0 + off) * NDF, DROWS)], sem).wait()
            return carry
        lax.fori_loop(0, NDCH, d_chunk, 0)

    return k(fsc_t, fdc_seg, ws_cols, wd_flat)


BB = 64


def _tc_assemble(stat_emb, stat_real, dyn_emb, dyn_real):
    def body(se_ref, sr_ref, de_ref, dr_ref, o_ref):
        stat = jnp.concatenate([se_ref[...], sr_ref[...]], axis=-1)
        statb = jnp.broadcast_to(stat[None, :, :], (T, BB, CS + NRS))
        o_ref[...] = jnp.concatenate(
            [statb, de_ref[...], dr_ref[...]], axis=-1)

    return pl.pallas_call(
        body,
        grid=(B // BB,),
        in_specs=[
            pl.BlockSpec((BB, CS), lambda i: (i, 0)),
            pl.BlockSpec((BB, NRS), lambda i: (i, 0)),
            pl.BlockSpec((T, BB, NDF * D), lambda i: (0, i, 0)),
            pl.BlockSpec((T, BB, NRD), lambda i: (0, i, 0)),
        ],
        out_specs=pl.BlockSpec((T, BB, COUT), lambda i: (0, i, 0)),
        out_shape=jax.ShapeDtypeStruct((T, B, COUT), jnp.float32),
    )(stat_emb, stat_real, dyn_emb, dyn_real)


def kernel(feat_static_cat, feat_static_real, feat_dynamic_cat,
           feat_dynamic_real, W_static, W_dynamic):
    ws_cols = jnp.transpose(W_static, (0, 2, 1)).reshape(NSF * D * V)
    wd_flat = W_dynamic.reshape(NDF * V, D)
    fsc_t = jnp.transpose(feat_static_cat.astype(jnp.int32),
                          (1, 0)).reshape(NSF * B)
    fdc_seg = jnp.transpose(feat_dynamic_cat.astype(jnp.int32),
                            (2, 1, 0)).reshape(NDF, T * B)
    out_stat, out_dyn = _sc_gather(fsc_t, fdc_seg, ws_cols, wd_flat)
    fdr_t = jnp.transpose(feat_dynamic_real, (1, 0, 2))
    out_t = _tc_assemble(
        out_stat.reshape(B, CS),
        feat_static_real,
        out_dyn.reshape(T, B, NDF * D),
        fdr_t,
    )
    return jnp.transpose(out_t, (1, 0, 2))


# speedup vs baseline: 6.7219x; 1.0066x over previous
import functools

import jax
import jax.numpy as jnp
from jax import lax
from jax.experimental import pallas as pl
from jax.experimental.pallas import tpu as pltpu
from jax.experimental.pallas import tpu_sc as plsc

B = 4096
T = 50
NSF = 26
NDF = 5
V = 100000
D = 16
NRS = 4
NRD = 8
CS = NSF * D
COUT = CS + NRS + NDF * D + NRD

NW = 32
BPW = B // NW
SCB = 32
SROWS = SCB * CS
NSCH = BPW // SCB

RPW = (T * B) // NW
DCR = 400
DROWS = DCR * NDF
DPAD = 2048
NDCH = RPW // DCR


def _sc_gather(fsc_t, fdc_seg, ws_cols, wd_flat):
    mesh = plsc.VectorSubcoreMesh(core_axis_name="c", subcore_axis_name="s")

    @functools.partial(
        pl.kernel,
        mesh=mesh,
        compiler_params=pltpu.CompilerParams(
            use_tc_tiling_on_sc=False, needs_layout_passes=False),
        out_type=(
            jax.ShapeDtypeStruct((B * CS,), jnp.float32),
            jax.ShapeDtypeStruct((T * B * NDF, D), jnp.float32),
        ),
        scratch_types=[
            pltpu.VMEM((NSF * BPW,), jnp.int32),
            pltpu.VMEM((SROWS // 128, 128), jnp.int32),
            pltpu.VMEM((SROWS,), jnp.float32),
            pltpu.VMEM((NDF * DCR + 48,), jnp.int32),
            pltpu.VMEM((DPAD // 128, 128), jnp.int32),
            pltpu.VMEM((DPAD, D), jnp.float32),
            pltpu.SemaphoreType.DMA,
        ],
    )
    def k(fsc_h, fdc_h, wsc_h, wd_h, outs_h, outd_h,
          sbuf, sidx, sdst, dbuf, didx, ddst, sem):
        w = lax.axis_index("s") * 2 + lax.axis_index("c")
        iota = lax.iota(jnp.int32, 16)
        b0 = w * BPW
        r0 = w * RPW

        hs = [
            pltpu.async_copy(fsc_h.at[pl.ds(i * B + b0, BPW)],
                             sbuf.at[pl.ds(i * BPW, BPW)], sem)
            for i in range(NSF)
        ]
        for h in hs:
            h.wait()

        def s_chunk(ci, carry):
            def comp(q, c2):
                p = q * 16 + iota
                col = p % CS
                bl = ci * SCB + p // CS
                raw = plsc.load_gather(sbuf, [(col // D) * BPW + bl])
                sidx[q // 8, pl.ds((q % 8) * 16, 16)] = col * V + raw
                return c2
            lax.fori_loop(0, SROWS // 16, comp, 0)

            def s_gat(j, c2):
                pltpu.async_copy(wsc_h.at[sidx.at[j]],
                                 sdst.at[pl.ds(j * 128, 128)], sem)
                return c2
            lax.fori_loop(0, SROWS // 128, s_gat, 0)
            pltpu.make_async_copy(wsc_h.at[pl.ds(0, SROWS)], sdst,
                                  sem).wait()
            pltpu.async_copy(
                sdst, outs_h.at[pl.ds((b0 + ci * SCB) * CS, SROWS)],
                sem).wait()
            return carry
        lax.fori_loop(0, NSCH, s_chunk, 0)

        def d_chunk(ci, carry):
            off = ci * DCR
            hseg = [
                pltpu.async_copy(fdc_h.at[f, pl.ds(r0 + off, DCR)],
                                 dbuf.at[pl.ds(f * DCR, DCR)], sem)
                for f in range(NDF)
            ]
            for h in hseg:
                h.wait()

            def comp(q, c2):
                p = q * 16 + iota
                rr = jnp.minimum(p // NDF, DCR - 1)
                f = p % NDF
                raw = plsc.load_gather(dbuf, [f * DCR + rr])
                didx[q // 8, pl.ds((q % 8) * 16, 16)] = f * V + raw
                return c2
            lax.fori_loop(0, DPAD // 16, comp, 0)

            def d_gat(j, c2):
                pltpu.async_copy(wd_h.at[didx.at[j]],
                                 ddst.at[pl.ds(j * 128, 128)], sem)
                return c2
            lax.fori_loop(0, DPAD // 128, d_gat, 0)
            pltpu.make_async_copy(wd_h.at[pl.ds(0, DPAD)], ddst,
                                  sem).wait()
            pltpu.async_copy(
                ddst.at[pl.ds(0, DROWS)],
                outd_h.at[pl.ds((r0 + off) * NDF, DROWS)], sem).wait()
            return carry
        lax.fori_loop(0, NDCH, d_chunk, 0)

    return k(fsc_t, fdc_seg, ws_cols, wd_flat)


def _tc_assemble(stat_emb, stat_real, dyn_emb, dyn_real):
    def body(se_ref, sr_ref, de_ref, dr_ref, o_ref):
        o_ref[...] = jnp.concatenate(
            [se_ref[...][None], sr_ref[...][None],
             de_ref[...], dr_ref[...]], axis=-1)

    return pl.pallas_call(
        body,
        grid=(T,),
        in_specs=[
            pl.BlockSpec((B, CS), lambda i: (0, 0)),
            pl.BlockSpec((B, NRS), lambda i: (0, 0)),
            pl.BlockSpec((1, B, NDF * D), lambda i: (i, 0, 0)),
            pl.BlockSpec((1, B, NRD), lambda i: (i, 0, 0)),
        ],
        out_specs=pl.BlockSpec((1, B, COUT), lambda i: (i, 0, 0)),
        out_shape=jax.ShapeDtypeStruct((T, B, COUT), jnp.float32),
    )(stat_emb, stat_real, dyn_emb, dyn_real)


def kernel(feat_static_cat, feat_static_real, feat_dynamic_cat,
           feat_dynamic_real, W_static, W_dynamic):
    ws_cols = jnp.transpose(W_static, (0, 2, 1)).reshape(NSF * D * V)
    wd_flat = W_dynamic.reshape(NDF * V, D)
    fsc_t = jnp.transpose(feat_static_cat.astype(jnp.int32),
                          (1, 0)).reshape(NSF * B)
    fdc_seg = jnp.transpose(feat_dynamic_cat.astype(jnp.int32),
                            (2, 1, 0)).reshape(NDF, T * B)
    out_stat, out_dyn = _sc_gather(fsc_t, fdc_seg, ws_cols, wd_flat)
    fdr_t = jnp.transpose(feat_dynamic_real, (1, 0, 2))
    out_t = _tc_assemble(
        out_stat.reshape(B, CS),
        feat_static_real,
        out_dyn.reshape(T, B, NDF * D),
        fdr_t,
    )
    return jnp.transpose(out_t, (1, 0, 2))
